# baseline (device time: 445015 ns/iter reference)
import jax
import jax.numpy as jnp
from jax import lax
from jax.experimental import pallas as pl
from jax.experimental.pallas import tpu as pltpu

N_DEV = 16
N_HOPS = N_DEV - 1
B = 4096
BP = B // N_DEV
D = 256
HP = 512


def kernel(x, Win0, Wout0, Win1, Wout1, Win2, Wout2):
    def body(x_ref, win0_ref, wout0_ref, win1_ref, wout1_ref, win2_ref,
             wout2_ref, out_ref, xfull_ref, acc_ref, send_buf, recv_bufs,
             send_sems, recv_sems):
        my = lax.axis_index("i")
        right = lax.rem(my + 1, N_DEV)
        left = lax.rem(my + N_DEV - 1, N_DEV)

        barrier = pltpu.get_barrier_semaphore()

        def neighbor_barrier():
            pl.semaphore_signal(barrier, inc=1, device_id=(left,),
                                device_id_type=pl.DeviceIdType.MESH)
            pl.semaphore_signal(barrier, inc=1, device_id=(right,),
                                device_id_type=pl.DeviceIdType.MESH)
            pl.semaphore_wait(barrier, 2)

        def all_gather():
            neighbor_barrier()
            for h in range(N_HOPS):
                c = lax.rem(my - h + N_DEV, N_DEV)
                rdma = pltpu.make_async_remote_copy(
                    src_ref=xfull_ref.at[c],
                    dst_ref=xfull_ref.at[c],
                    send_sem=send_sems.at[h],
                    recv_sem=recv_sems.at[h],
                    device_id=(right,),
                    device_id_type=pl.DeviceIdType.MESH,
                )
                rdma.start()
                rdma.wait()

        def layer(win_ref, wout_ref):
            for c in range(N_DEV):
                h = jnp.maximum(
                    jnp.dot(xfull_ref[c], win_ref[...],
                            preferred_element_type=jnp.float32),
                    0.0,
                )
                acc_ref[c] = jnp.dot(h, wout_ref[...],
                                     preferred_element_type=jnp.float32)

        def reduce_scatter():
            neighbor_barrier()
            val = acc_ref[lax.rem(my + N_DEV - 1, N_DEV)]
            for s in range(N_HOPS):
                send_buf[...] = val
                rdma = pltpu.make_async_remote_copy(
                    src_ref=send_buf,
                    dst_ref=recv_bufs.at[s],
                    send_sem=send_sems.at[s],
                    recv_sem=recv_sems.at[s],
                    device_id=(right,),
                    device_id_type=pl.DeviceIdType.MESH,
                )
                rdma.start()
                rdma.wait()
                c_r = lax.rem(my - s - 2 + 2 * N_DEV, N_DEV)
                val = recv_bufs[s] + acc_ref[c_r]
            return val

        xfull_ref[my] = x_ref[...]
        all_gather()
        for win_ref, wout_ref, last in (
            (win0_ref, wout0_ref, False),
            (win1_ref, wout1_ref, False),
            (win2_ref, wout2_ref, True),
        ):
            layer(win_ref, wout_ref)
            val = reduce_scatter()
            if last:
                out_ref[...] = val
            else:
                xfull_ref[my] = val
                all_gather()

    return pl.pallas_call(
        body,
        out_shape=jax.ShapeDtypeStruct((BP, D), jnp.float32),
        in_specs=[pl.BlockSpec(memory_space=pltpu.VMEM)] * 7,
        out_specs=pl.BlockSpec(memory_space=pltpu.VMEM),
        scratch_shapes=[
            pltpu.VMEM((N_DEV, BP, D), jnp.float32),
            pltpu.VMEM((N_DEV, BP, D), jnp.float32),
            pltpu.VMEM((BP, D), jnp.float32),
            pltpu.VMEM((N_HOPS, BP, D), jnp.float32),
            pltpu.SemaphoreType.DMA((N_HOPS,)),
            pltpu.SemaphoreType.DMA((N_HOPS,)),
        ],
        compiler_params=pltpu.CompilerParams(collective_id=0),
    )(x, Win0, Wout0, Win1, Wout1, Win2, Wout2)


# device time: 249203 ns/iter; 1.7858x vs baseline; 1.7858x over previous
import jax
import jax.numpy as jnp
from jax import lax
from jax.experimental import pallas as pl
from jax.experimental.pallas import tpu as pltpu

N_DEV = 16
N_HOPS = N_DEV - 1
BP = 256
HB = BP // 2
D = 256
HP = 512


def kernel(x, Win0, Wout0, Win1, Wout1, Win2, Wout2):
    def body(x_ref, win0_ref, wout0_ref, win1_ref, wout1_ref, win2_ref,
             wout2_ref, out_ref,
             xR, xL, pRmy, pLmy, sR, sL, rR, rL,
             agR_ss, agR_rs, agL_ss, agL_rs,
             rsR_ss, rsR_rs, rsL_ss, rsL_rs):
        my = lax.axis_index("i")
        right = lax.rem(my + 1, N_DEV)
        left = lax.rem(my + N_DEV - 1, N_DEV)
        barrier = pltpu.get_barrier_semaphore()

        def idx(v):
            return lax.rem(v + 2 * N_DEV, N_DEV)

        def layer(win_ref, wout_ref, last):
            pl.semaphore_signal(barrier, inc=1, device_id=(left,),
                                device_id_type=pl.DeviceIdType.MESH)
            pl.semaphore_signal(barrier, inc=1, device_id=(right,),
                                device_id_type=pl.DeviceIdType.MESH)
            pl.semaphore_wait(barrier, 2)

            ag_descs = []
            rs_descs = []
            for k in range(N_DEV):
                if k >= 1:
                    ag_descs[k - 1][0].wait_recv()
                    ag_descs[k - 1][1].wait_recv()
                cR = idx(my - k)
                cL = idx(my + k)
                if k < N_HOPS:
                    dR = pltpu.make_async_remote_copy(
                        src_ref=xR.at[cR], dst_ref=xR.at[cR],
                        send_sem=agR_ss.at[k], recv_sem=agR_rs.at[k],
                        device_id=(right,),
                        device_id_type=pl.DeviceIdType.MESH)
                    dR.start()
                    dL = pltpu.make_async_remote_copy(
                        src_ref=xL.at[cL], dst_ref=xL.at[cL],
                        send_sem=agL_ss.at[k], recv_sem=agL_rs.at[k],
                        device_id=(left,),
                        device_id_type=pl.DeviceIdType.MESH)
                    dL.start()
                    ag_descs.append((dR, dL))
                hR = jnp.maximum(
                    jnp.dot(xR[cR], win_ref[...],
                            preferred_element_type=jnp.float32), 0.0)
                pR = jnp.dot(hR, wout_ref[...],
                             preferred_element_type=jnp.float32)
                hL = jnp.maximum(
                    jnp.dot(xL[cL], win_ref[...],
                            preferred_element_type=jnp.float32), 0.0)
                pL = jnp.dot(hL, wout_ref[...],
                             preferred_element_type=jnp.float32)
                if k == 0:
                    pRmy[...] = pR
                    pLmy[...] = pL
                else:
                    t = k - 1
                    if t >= 1:
                        rs_descs[t - 1][0].wait_recv()
                        rs_descs[t - 1][1].wait_recv()
                        SR = pR + rR[t - 1]
                        SL = pL + rL[t - 1]
                    else:
                        SR = pR
                        SL = pL
                    sR[t] = SR
                    sL[t] = SL
                    eR = pltpu.make_async_remote_copy(
                        src_ref=sR.at[t], dst_ref=rR.at[t],
                        send_sem=rsR_ss.at[t], recv_sem=rsR_rs.at[t],
                        device_id=(right,),
                        device_id_type=pl.DeviceIdType.MESH)
                    eR.start()
                    eL = pltpu.make_async_remote_copy(
                        src_ref=sL.at[t], dst_ref=rL.at[t],
                        send_sem=rsL_ss.at[t], recv_sem=rsL_rs.at[t],
                        device_id=(left,),
                        device_id_type=pl.DeviceIdType.MESH)
                    eL.start()
                    rs_descs.append((eR, eL))

            rs_descs[N_HOPS - 1][0].wait_recv()
            rs_descs[N_HOPS - 1][1].wait_recv()
            outR = pRmy[...] + rR[N_HOPS - 1]
            outL = pLmy[...] + rL[N_HOPS - 1]
            for dR, dL in ag_descs:
                dR.wait_send()
                dL.wait_send()
            for eR, eL in rs_descs:
                eR.wait_send()
                eL.wait_send()
            if last:
                out_ref[0:HB, :] = outR
                out_ref[HB:BP, :] = outL
            else:
                xR[my] = outR
                xL[my] = outL

        xR[my] = x_ref[0:HB, :]
        xL[my] = x_ref[HB:BP, :]
        layer(win0_ref, wout0_ref, False)
        layer(win1_ref, wout1_ref, False)
        layer(win2_ref, wout2_ref, True)

    return pl.pallas_call(
        body,
        out_shape=jax.ShapeDtypeStruct((BP, D), jnp.float32),
        in_specs=[pl.BlockSpec(memory_space=pltpu.VMEM)] * 7,
        out_specs=pl.BlockSpec(memory_space=pltpu.VMEM),
        scratch_shapes=[
            pltpu.VMEM((N_DEV, HB, D), jnp.float32),
            pltpu.VMEM((N_DEV, HB, D), jnp.float32),
            pltpu.VMEM((HB, D), jnp.float32),
            pltpu.VMEM((HB, D), jnp.float32),
            pltpu.VMEM((N_HOPS, HB, D), jnp.float32),
            pltpu.VMEM((N_HOPS, HB, D), jnp.float32),
            pltpu.VMEM((N_HOPS, HB, D), jnp.float32),
            pltpu.VMEM((N_HOPS, HB, D), jnp.float32),
            pltpu.SemaphoreType.DMA((N_HOPS,)),
            pltpu.SemaphoreType.DMA((N_HOPS,)),
            pltpu.SemaphoreType.DMA((N_HOPS,)),
            pltpu.SemaphoreType.DMA((N_HOPS,)),
            pltpu.SemaphoreType.DMA((N_HOPS,)),
            pltpu.SemaphoreType.DMA((N_HOPS,)),
            pltpu.SemaphoreType.DMA((N_HOPS,)),
            pltpu.SemaphoreType.DMA((N_HOPS,)),
        ],
        compiler_params=pltpu.CompilerParams(collective_id=0),
    )(x, Win0, Wout0, Win1, Wout1, Win2, Wout2)


# device time: 178885 ns/iter; 2.4877x vs baseline; 1.3931x over previous
import jax
import jax.numpy as jnp
from jax import lax
from jax.experimental import pallas as pl
from jax.experimental.pallas import tpu as pltpu

N_DEV = 16
N_HOPS = N_DEV - 1
BP = 256
HB = BP // 2
D = 256
HP = 512

PERM = [0, 4, 8, 12, 13, 9, 5, 1, 2, 6, 10, 14, 15, 11, 7, 3]
POS = [0] * N_DEV
for _q, _l in enumerate(PERM):
    POS[_l] = _q


def kernel(x, Win0, Wout0, Win1, Wout1, Win2, Wout2):
    def body(x_ref, win0_ref, wout0_ref, win1_ref, wout1_ref, win2_ref,
             wout2_ref, out_ref,
             xR, xL, pRmy, pLmy, sR, sL, rR, rL,
             agR_ss, agR_rs, agL_ss, agL_rs,
             rsR_ss, rsR_rs, rsL_ss, rsL_rs):
        my = lax.axis_index("i")
        p = jnp.int32(0)
        for l in range(N_DEV):
            p = jnp.where(my == l, jnp.int32(POS[l]), p)
        right = jnp.int32(0)
        left = jnp.int32(0)
        for q in range(N_DEV):
            right = jnp.where(p == q, jnp.int32(PERM[(q + 1) % N_DEV]), right)
            left = jnp.where(p == q, jnp.int32(PERM[(q - 1) % N_DEV]), left)
        barrier = pltpu.get_barrier_semaphore()

        def idx(v):
            return lax.rem(v + 2 * N_DEV, N_DEV)

        def layer(win_ref, wout_ref, last):
            pl.semaphore_signal(barrier, inc=1, device_id=(left,),
                                device_id_type=pl.DeviceIdType.MESH)
            pl.semaphore_signal(barrier, inc=1, device_id=(right,),
                                device_id_type=pl.DeviceIdType.MESH)
            pl.semaphore_wait(barrier, 2)

            ag_descs = []
            rs_descs = []
            for k in range(N_DEV):
                if k >= 1:
                    ag_descs[k - 1][0].wait_recv()
                    ag_descs[k - 1][1].wait_recv()
                cR = idx(p - k)
                cL = idx(p + k)
                if k < N_HOPS:
                    dR = pltpu.make_async_remote_copy(
                        src_ref=xR.at[cR], dst_ref=xR.at[cR],
                        send_sem=agR_ss.at[k], recv_sem=agR_rs.at[k],
                        device_id=(right,),
                        device_id_type=pl.DeviceIdType.MESH)
                    dR.start()
                    dL = pltpu.make_async_remote_copy(
                        src_ref=xL.at[cL], dst_ref=xL.at[cL],
                        send_sem=agL_ss.at[k], recv_sem=agL_rs.at[k],
                        device_id=(left,),
                        device_id_type=pl.DeviceIdType.MESH)
                    dL.start()
                    ag_descs.append((dR, dL))
                hR = jnp.maximum(
                    jnp.dot(xR[cR], win_ref[...],
                            preferred_element_type=jnp.float32), 0.0)
                pR = jnp.dot(hR, wout_ref[...],
                             preferred_element_type=jnp.float32)
                hL = jnp.maximum(
                    jnp.dot(xL[cL], win_ref[...],
                            preferred_element_type=jnp.float32), 0.0)
                pL = jnp.dot(hL, wout_ref[...],
                             preferred_element_type=jnp.float32)
                if k == 0:
                    pRmy[...] = pR
                    pLmy[...] = pL
                else:
                    t = k - 1
                    if t >= 1:
                        rs_descs[t - 1][0].wait_recv()
                        rs_descs[t - 1][1].wait_recv()
                        SR = pR + rR[t - 1]
                        SL = pL + rL[t - 1]
                    else:
                        SR = pR
                        SL = pL
                    sR[t] = SR
                    sL[t] = SL
                    eR = pltpu.make_async_remote_copy(
                        src_ref=sR.at[t], dst_ref=rR.at[t],
                        send_sem=rsR_ss.at[t], recv_sem=rsR_rs.at[t],
                        device_id=(right,),
                        device_id_type=pl.DeviceIdType.MESH)
                    eR.start()
                    eL = pltpu.make_async_remote_copy(
                        src_ref=sL.at[t], dst_ref=rL.at[t],
                        send_sem=rsL_ss.at[t], recv_sem=rsL_rs.at[t],
                        device_id=(left,),
                        device_id_type=pl.DeviceIdType.MESH)
                    eL.start()
                    rs_descs.append((eR, eL))

            rs_descs[N_HOPS - 1][0].wait_recv()
            rs_descs[N_HOPS - 1][1].wait_recv()
            outR = pRmy[...] + rR[N_HOPS - 1]
            outL = pLmy[...] + rL[N_HOPS - 1]
            for dR, dL in ag_descs:
                dR.wait_send()
                dL.wait_send()
            for eR, eL in rs_descs:
                eR.wait_send()
                eL.wait_send()
            if last:
                out_ref[0:HB, :] = outR
                out_ref[HB:BP, :] = outL
            else:
                xR[p] = outR
                xL[p] = outL

        xR[p] = x_ref[0:HB, :]
        xL[p] = x_ref[HB:BP, :]
        layer(win0_ref, wout0_ref, False)
        layer(win1_ref, wout1_ref, False)
        layer(win2_ref, wout2_ref, True)

    return pl.pallas_call(
        body,
        out_shape=jax.ShapeDtypeStruct((BP, D), jnp.float32),
        in_specs=[pl.BlockSpec(memory_space=pltpu.VMEM)] * 7,
        out_specs=pl.BlockSpec(memory_space=pltpu.VMEM),
        scratch_shapes=[
            pltpu.VMEM((N_DEV, HB, D), jnp.float32),
            pltpu.VMEM((N_DEV, HB, D), jnp.float32),
            pltpu.VMEM((HB, D), jnp.float32),
            pltpu.VMEM((HB, D), jnp.float32),
            pltpu.VMEM((N_HOPS, HB, D), jnp.float32),
            pltpu.VMEM((N_HOPS, HB, D), jnp.float32),
            pltpu.VMEM((N_HOPS, HB, D), jnp.float32),
            pltpu.VMEM((N_HOPS, HB, D), jnp.float32),
            pltpu.SemaphoreType.DMA((N_HOPS,)),
            pltpu.SemaphoreType.DMA((N_HOPS,)),
            pltpu.SemaphoreType.DMA((N_HOPS,)),
            pltpu.SemaphoreType.DMA((N_HOPS,)),
            pltpu.SemaphoreType.DMA((N_HOPS,)),
            pltpu.SemaphoreType.DMA((N_HOPS,)),
            pltpu.SemaphoreType.DMA((N_HOPS,)),
            pltpu.SemaphoreType.DMA((N_HOPS,)),
        ],
        compiler_params=pltpu.CompilerParams(collective_id=0),
    )(x, Win0, Wout0, Win1, Wout1, Win2, Wout2)


# device time: 156546 ns/iter; 2.8427x vs baseline; 1.1427x over previous
import jax
import jax.numpy as jnp
from jax import lax
from jax.experimental import pallas as pl
from jax.experimental.pallas import tpu as pltpu

N_DEV = 16
N_HOPS = N_DEV - 1
BP = 256
HB = BP // 2
NS = 2
SB = HB // NS
D = 256
HP = 512

PERM = [0, 4, 8, 12, 13, 9, 5, 1, 2, 6, 10, 14, 15, 11, 7, 3]
POS = [0] * N_DEV
for _q, _l in enumerate(PERM):
    POS[_l] = _q


def kernel(x, Win0, Wout0, Win1, Wout1, Win2, Wout2):
    def body(x_ref, win0_ref, wout0_ref, win1_ref, wout1_ref, win2_ref,
             wout2_ref, out_ref,
             xR, xL, pRmy, pLmy, sR, sL, rR, rL,
             agR_ss, agR_rs, agL_ss, agL_rs,
             rsR_ss, rsR_rs, rsL_ss, rsL_rs):
        my = lax.axis_index("i")
        p = jnp.int32(0)
        for l in range(N_DEV):
            p = jnp.where(my == l, jnp.int32(POS[l]), p)
        right = jnp.int32(0)
        left = jnp.int32(0)
        for q in range(N_DEV):
            right = jnp.where(p == q, jnp.int32(PERM[(q + 1) % N_DEV]), right)
            left = jnp.where(p == q, jnp.int32(PERM[(q - 1) % N_DEV]), left)
        barrier = pltpu.get_barrier_semaphore()

        def idx(v):
            return lax.rem(v + 2 * N_DEV, N_DEV)

        def ag_send(buf, c, ss, rs, k, s, dev):
            d = pltpu.make_async_remote_copy(
                src_ref=buf.at[c, s], dst_ref=buf.at[c, s],
                send_sem=ss.at[k, s], recv_sem=rs.at[k, s],
                device_id=(dev,), device_id_type=pl.DeviceIdType.MESH)
            d.start()
            return d

        def rs_send(sbuf, rbuf, ss, rs, t, s, dev):
            d = pltpu.make_async_remote_copy(
                src_ref=sbuf.at[t, s], dst_ref=rbuf.at[t, s],
                send_sem=ss.at[t, s], recv_sem=rs.at[t, s],
                device_id=(dev,), device_id_type=pl.DeviceIdType.MESH)
            d.start()
            return d

        def layer(win_ref, wout_ref, last):
            pl.semaphore_signal(barrier, inc=1, device_id=(left,),
                                device_id_type=pl.DeviceIdType.MESH)
            pl.semaphore_signal(barrier, inc=1, device_id=(right,),
                                device_id_type=pl.DeviceIdType.MESH)
            pl.semaphore_wait(barrier, 2)

            ag_descs = []
            rs_descs = []
            for k in range(N_DEV):
                cR = idx(p - k)
                cL = idx(p + k)
                if k == 0:
                    ds = [ag_send(xR, cR, agR_ss, agR_rs, 0, s, right)
                          for s in range(NS)]
                    ds += [ag_send(xL, cL, agL_ss, agL_rs, 0, s, left)
                           for s in range(NS)]
                    ag_descs.append(ds)
                else:
                    prev = ag_descs[k - 1]
                    ds = []
                    for s in range(NS):
                        prev[s].wait_recv()
                        if k < N_HOPS:
                            ds.append(ag_send(xR, cR, agR_ss, agR_rs,
                                              k, s, right))
                    for s in range(NS):
                        prev[NS + s].wait_recv()
                        if k < N_HOPS:
                            ds.append(ag_send(xL, cL, agL_ss, agL_rs,
                                              k, s, left))
                    if k < N_HOPS:
                        ag_descs.append(ds)
                hR = jnp.maximum(
                    jnp.dot(xR[cR].reshape(HB, D), win_ref[...],
                            preferred_element_type=jnp.float32), 0.0)
                pR = jnp.dot(hR, wout_ref[...],
                             preferred_element_type=jnp.float32)
                hL = jnp.maximum(
                    jnp.dot(xL[cL].reshape(HB, D), win_ref[...],
                            preferred_element_type=jnp.float32), 0.0)
                pL = jnp.dot(hL, wout_ref[...],
                             preferred_element_type=jnp.float32)
                if k == 0:
                    pRmy[...] = pR
                    pLmy[...] = pL
                else:
                    t = k - 1
                    es = []
                    if t >= 1:
                        prev = rs_descs[t - 1]
                        for s in range(NS):
                            prev[s].wait_recv()
                            sR[t, s] = (pR[s * SB:(s + 1) * SB, :]
                                        + rR[t - 1, s])
                            es.append(rs_send(sR, rR, rsR_ss, rsR_rs,
                                              t, s, right))
                        for s in range(NS):
                            prev[NS + s].wait_recv()
                            sL[t, s] = (pL[s * SB:(s + 1) * SB, :]
                                        + rL[t - 1, s])
                            es.append(rs_send(sL, rL, rsL_ss, rsL_rs,
                                              t, s, left))
                    else:
                        for s in range(NS):
                            sR[t, s] = pR[s * SB:(s + 1) * SB, :]
                            es.append(rs_send(sR, rR, rsR_ss, rsR_rs,
                                              t, s, right))
                        for s in range(NS):
                            sL[t, s] = pL[s * SB:(s + 1) * SB, :]
                            es.append(rs_send(sL, rL, rsL_ss, rsL_rs,
                                              t, s, left))
                    rs_descs.append(es)

            fin = rs_descs[N_HOPS - 1]
            for d in fin:
                d.wait_recv()
            outR = pRmy[...] + rR[N_HOPS - 1].reshape(HB, D)
            outL = pLmy[...] + rL[N_HOPS - 1].reshape(HB, D)
            for ds in ag_descs:
                for d in ds:
                    d.wait_send()
            for es in rs_descs:
                for d in es:
                    d.wait_send()
            if last:
                out_ref[0:HB, :] = outR
                out_ref[HB:BP, :] = outL
            else:
                xR[p] = outR.reshape(NS, SB, D)
                xL[p] = outL.reshape(NS, SB, D)

        xR[p] = x_ref[0:HB, :].reshape(NS, SB, D)
        xL[p] = x_ref[HB:BP, :].reshape(NS, SB, D)
        layer(win0_ref, wout0_ref, False)
        layer(win1_ref, wout1_ref, False)
        layer(win2_ref, wout2_ref, True)

    return pl.pallas_call(
        body,
        out_shape=jax.ShapeDtypeStruct((BP, D), jnp.float32),
        in_specs=[pl.BlockSpec(memory_space=pltpu.VMEM)] * 7,
        out_specs=pl.BlockSpec(memory_space=pltpu.VMEM),
        scratch_shapes=[
            pltpu.VMEM((N_DEV, NS, SB, D), jnp.float32),
            pltpu.VMEM((N_DEV, NS, SB, D), jnp.float32),
            pltpu.VMEM((HB, D), jnp.float32),
            pltpu.VMEM((HB, D), jnp.float32),
            pltpu.VMEM((N_HOPS, NS, SB, D), jnp.float32),
            pltpu.VMEM((N_HOPS, NS, SB, D), jnp.float32),
            pltpu.VMEM((N_HOPS, NS, SB, D), jnp.float32),
            pltpu.VMEM((N_HOPS, NS, SB, D), jnp.float32),
            pltpu.SemaphoreType.DMA((N_HOPS, NS)),
            pltpu.SemaphoreType.DMA((N_HOPS, NS)),
            pltpu.SemaphoreType.DMA((N_HOPS, NS)),
            pltpu.SemaphoreType.DMA((N_HOPS, NS)),
            pltpu.SemaphoreType.DMA((N_HOPS, NS)),
            pltpu.SemaphoreType.DMA((N_HOPS, NS)),
            pltpu.SemaphoreType.DMA((N_HOPS, NS)),
            pltpu.SemaphoreType.DMA((N_HOPS, NS)),
        ],
        compiler_params=pltpu.CompilerParams(collective_id=0),
    )(x, Win0, Wout0, Win1, Wout1, Win2, Wout2)


# device time: 139302 ns/iter; 3.1946x vs baseline; 1.1238x over previous
import jax
import jax.numpy as jnp
from jax import lax
from jax.experimental import pallas as pl
from jax.experimental.pallas import tpu as pltpu

N_DEV = 16
N_HOPS = N_DEV - 1
BP = 256
HB = BP // 2
NS = 2
SB = HB // NS
D = 256
HP = 512

PERM = [0, 4, 8, 12, 13, 9, 5, 1, 2, 6, 10, 14, 15, 11, 7, 3]
POS = [0] * N_DEV
for _q, _l in enumerate(PERM):
    POS[_l] = _q


def kernel(x, Win0, Wout0, Win1, Wout1, Win2, Wout2):
    def body(x_ref, win0_ref, wout0_ref, win1_ref, wout1_ref, win2_ref,
             wout2_ref, out_ref,
             xR, xL, pRmy, pLmy, sR, sL, rR, rL, wbin, wbout,
             agR_ss, agR_rs, agL_ss, agL_rs,
             rsR_ss, rsR_rs, rsL_ss, rsL_rs):
        my = lax.axis_index("i")
        p = jnp.int32(0)
        for l in range(N_DEV):
            p = jnp.where(my == l, jnp.int32(POS[l]), p)
        right = jnp.int32(0)
        left = jnp.int32(0)
        for q in range(N_DEV):
            right = jnp.where(p == q, jnp.int32(PERM[(q + 1) % N_DEV]), right)
            left = jnp.where(p == q, jnp.int32(PERM[(q - 1) % N_DEV]), left)
        barrier = pltpu.get_barrier_semaphore()

        def idx(v):
            return lax.rem(v + 2 * N_DEV, N_DEV)

        def ag_send(buf, c, ss, rs, k, s, dev):
            d = pltpu.make_async_remote_copy(
                src_ref=buf.at[c, s], dst_ref=buf.at[c, s],
                send_sem=ss.at[k, s], recv_sem=rs.at[k, s],
                device_id=(dev,), device_id_type=pl.DeviceIdType.MESH)
            d.start()
            return d

        def rs_send(sbuf, rbuf, ss, rs, t, s, dev):
            d = pltpu.make_async_remote_copy(
                src_ref=sbuf.at[t, s], dst_ref=rbuf.at[t, s],
                send_sem=ss.at[t, s], recv_sem=rs.at[t, s],
                device_id=(dev,), device_id_type=pl.DeviceIdType.MESH)
            d.start()
            return d

        def layer(win_ref, wout_ref, last):
            wbin[...] = win_ref[...].astype(jnp.bfloat16)
            wbout[...] = wout_ref[...].astype(jnp.bfloat16)
            pl.semaphore_signal(barrier, inc=1, device_id=(left,),
                                device_id_type=pl.DeviceIdType.MESH)
            pl.semaphore_signal(barrier, inc=1, device_id=(right,),
                                device_id_type=pl.DeviceIdType.MESH)
            pl.semaphore_wait(barrier, 2)

            ag_descs = []
            rs_descs = []
            for k in range(N_DEV):
                cR = idx(p - k)
                cL = idx(p + k)
                if k == 0:
                    ds = [ag_send(xR, cR, agR_ss, agR_rs, 0, s, right)
                          for s in range(NS)]
                    ds += [ag_send(xL, cL, agL_ss, agL_rs, 0, s, left)
                           for s in range(NS)]
                    ag_descs.append(ds)
                else:
                    prev = ag_descs[k - 1]
                    ds = []
                    for s in range(NS):
                        prev[s].wait_recv()
                        if k < N_HOPS:
                            ds.append(ag_send(xR, cR, agR_ss, agR_rs,
                                              k, s, right))
                    for s in range(NS):
                        prev[NS + s].wait_recv()
                        if k < N_HOPS:
                            ds.append(ag_send(xL, cL, agL_ss, agL_rs,
                                              k, s, left))
                    if k < N_HOPS:
                        ag_descs.append(ds)
                hR = jnp.maximum(
                    jnp.dot(xR[cR].reshape(HB, D), wbin[...],
                            preferred_element_type=jnp.float32), 0.0)
                pR = jnp.dot(hR.astype(jnp.bfloat16), wbout[...],
                             preferred_element_type=jnp.float32)
                hL = jnp.maximum(
                    jnp.dot(xL[cL].reshape(HB, D), wbin[...],
                            preferred_element_type=jnp.float32), 0.0)
                pL = jnp.dot(hL.astype(jnp.bfloat16), wbout[...],
                             preferred_element_type=jnp.float32)
                if k == 0:
                    pRmy[...] = pR
                    pLmy[...] = pL
                else:
                    t = k - 1
                    es = []
                    if t >= 1:
                        prev = rs_descs[t - 1]
                        for s in range(NS):
                            prev[s].wait_recv()
                            sR[t, s] = (pR[s * SB:(s + 1) * SB, :]
                                        + rR[t - 1, s])
                            es.append(rs_send(sR, rR, rsR_ss, rsR_rs,
                                              t, s, right))
                        for s in range(NS):
                            prev[NS + s].wait_recv()
                            sL[t, s] = (pL[s * SB:(s + 1) * SB, :]
                                        + rL[t - 1, s])
                            es.append(rs_send(sL, rL, rsL_ss, rsL_rs,
                                              t, s, left))
                    else:
                        for s in range(NS):
                            sR[t, s] = pR[s * SB:(s + 1) * SB, :]
                            es.append(rs_send(sR, rR, rsR_ss, rsR_rs,
                                              t, s, right))
                        for s in range(NS):
                            sL[t, s] = pL[s * SB:(s + 1) * SB, :]
                            es.append(rs_send(sL, rL, rsL_ss, rsL_rs,
                                              t, s, left))
                    rs_descs.append(es)

            fin = rs_descs[N_HOPS - 1]
            for d in fin:
                d.wait_recv()
            outR = pRmy[...] + rR[N_HOPS - 1].reshape(HB, D)
            outL = pLmy[...] + rL[N_HOPS - 1].reshape(HB, D)
            for ds in ag_descs:
                for d in ds:
                    d.wait_send()
            for es in rs_descs:
                for d in es:
                    d.wait_send()
            if last:
                out_ref[0:HB, :] = outR
                out_ref[HB:BP, :] = outL
            else:
                xR[p] = outR.astype(jnp.bfloat16).reshape(NS, SB, D)
                xL[p] = outL.astype(jnp.bfloat16).reshape(NS, SB, D)

        xR[p] = x_ref[0:HB, :].astype(jnp.bfloat16).reshape(NS, SB, D)
        xL[p] = x_ref[HB:BP, :].astype(jnp.bfloat16).reshape(NS, SB, D)
        layer(win0_ref, wout0_ref, False)
        layer(win1_ref, wout1_ref, False)
        layer(win2_ref, wout2_ref, True)

    return pl.pallas_call(
        body,
        out_shape=jax.ShapeDtypeStruct((BP, D), jnp.float32),
        in_specs=[pl.BlockSpec(memory_space=pltpu.VMEM)] * 7,
        out_specs=pl.BlockSpec(memory_space=pltpu.VMEM),
        scratch_shapes=[
            pltpu.VMEM((N_DEV, NS, SB, D), jnp.bfloat16),
            pltpu.VMEM((N_DEV, NS, SB, D), jnp.bfloat16),
            pltpu.VMEM((HB, D), jnp.float32),
            pltpu.VMEM((HB, D), jnp.float32),
            pltpu.VMEM((N_HOPS, NS, SB, D), jnp.float32),
            pltpu.VMEM((N_HOPS, NS, SB, D), jnp.float32),
            pltpu.VMEM((N_HOPS, NS, SB, D), jnp.float32),
            pltpu.VMEM((N_HOPS, NS, SB, D), jnp.float32),
            pltpu.VMEM((D, HP), jnp.bfloat16),
            pltpu.VMEM((HP, D), jnp.bfloat16),
            pltpu.SemaphoreType.DMA((N_HOPS, NS)),
            pltpu.SemaphoreType.DMA((N_HOPS, NS)),
            pltpu.SemaphoreType.DMA((N_HOPS, NS)),
            pltpu.SemaphoreType.DMA((N_HOPS, NS)),
            pltpu.SemaphoreType.DMA((N_HOPS, NS)),
            pltpu.SemaphoreType.DMA((N_HOPS, NS)),
            pltpu.SemaphoreType.DMA((N_HOPS, NS)),
            pltpu.SemaphoreType.DMA((N_HOPS, NS)),
        ],
        compiler_params=pltpu.CompilerParams(collective_id=0),
    )(x, Win0, Wout0, Win1, Wout1, Win2, Wout2)
